# Initial kernel scaffold; baseline (speedup 1.0000x reference)
import jax
import jax.numpy as jnp
from jax.experimental import pallas as pl


def kernel(glimpse__feature, glimpse__center, glimpse__batch, glimpse_member__local_pos, glimpse_member__log_mask, glimpse_member__glimpse_index, temperature, w1f1, b1f1, w1f2, b1f2, w1g1, b1g1, w1g2, b1g2, w2f1, b2f1, w2f2, b2f2, w2g1, b2g1, w2g2, b2g2, w3f1, b3f1, w3f2, b3f2, w3g1, b3g1, w3g2, b3g2, zp_w, zp_b):
    raise NotImplementedError("write your pallas kernel here")


# speedup vs baseline: 3.2462x; 3.2462x over previous
import functools
import jax
import jax.numpy as jnp
from jax import lax
from jax.experimental import pallas as pl
from jax.experimental.pallas import tpu as pltpu
from jax.experimental.pallas import tpu_sc as plsc

N = 10000
M = 160000
K = 32
RADIUS_MAX = 4.0

NC = 2
NS = 16
NW = NC * NS


_MBLK = 640
_NBLK = M // _MBLK
_CHW = 16


_NPAD = 10240
_NSTRIPE = _NPAD // NS


def _member_body(gidx_hbm, lm_hbm, px_hbm, py_hbm, zeros_hbm, out_hbm,
                 gv, lv, xv, yv, ix, vals, shared):
    cid = lax.axis_index("c")
    sid = lax.axis_index("s")
    wid = sid * NC + cid

    pltpu.sync_copy(zeros_hbm.at[pl.ds(sid * _NSTRIPE, _NSTRIPE)],
                    shared.at[pl.ds(sid * _NSTRIPE, _NSTRIPE)])
    plsc.subcore_barrier()

    pltpu.sync_copy(zeros_hbm.at[pl.ds(0, 16)], vals)

    lanes = lax.iota(jnp.int32, 16)
    zf = jnp.zeros((16,), jnp.float32)

    def blk_body(bi, carry):
        b = wid + bi * NW

        @pl.when(b < _NBLK)
        def _():
            base = b * _MBLK
            pltpu.sync_copy(gidx_hbm.at[pl.ds(base, _MBLK)], gv)
            pltpu.sync_copy(lm_hbm.at[pl.ds(base, _MBLK)], lv)
            pltpu.sync_copy(px_hbm.at[pl.ds(base, _MBLK)], xv)
            pltpu.sync_copy(py_hbm.at[pl.ds(base, _MBLK)], yv)

            def chunk_body(j, c2):
                sl = pl.ds(j * 16, 16)
                g = gv[sl]
                w = jnp.exp(lv[sl])
                wx = w * xv[sl]
                wy = w * yv[sl]
                ix[...] = g
                for r in range(16):
                    hit = lanes == r
                    vals[r, pl.ds(0, 16)] = jnp.where(hit, w, zf)
                    vals[r, pl.ds(16, 16)] = jnp.where(hit, wx, zf)
                    vals[r, pl.ds(32, 16)] = jnp.where(hit, wy, zf)
                pltpu.sync_copy(vals, shared.at[ix], add=True)
                return c2
            lax.fori_loop(0, _MBLK // 16, chunk_body, 0)
        return carry

    lax.fori_loop(0, (_NBLK + NW - 1) // NW, blk_body, 0)
    plsc.subcore_barrier()
    pltpu.sync_copy(shared.at[pl.ds(sid * _NSTRIPE, _NSTRIPE)],
                    out_hbm.at[pl.ds(cid * _NPAD + sid * _NSTRIPE, _NSTRIPE)])


def _member_segsum(gidx, lm, px, py):
    mesh = plsc.VectorSubcoreMesh(core_axis_name="c", subcore_axis_name="s")
    zeros = jnp.zeros((_NPAD, 128), jnp.float32)
    kern = functools.partial(
        pl.kernel, mesh=mesh,
        out_type=jax.ShapeDtypeStruct((NC * _NPAD, 128), jnp.float32),
        scratch_types=[
            pltpu.VMEM((_MBLK,), jnp.int32),
            pltpu.VMEM((_MBLK,), jnp.float32),
            pltpu.VMEM((_MBLK,), jnp.float32),
            pltpu.VMEM((_MBLK,), jnp.float32),
            pltpu.VMEM((16,), jnp.int32),
            pltpu.VMEM((16, 128), jnp.float32),
            pltpu.VMEM_SHARED((_NPAD, 128), jnp.float32),
        ],
    )(_member_body)
    return kern(gidx, lm, px, py, zeros)


_GBLK = 80


def _gather_body(table_hbm, idx_hbm, out_hbm, idx_v, rows_v, sem):
    wid = lax.axis_index("s") * NC + lax.axis_index("c")
    per_w = (N * K) // NW

    def body(i, carry):
        base = wid * per_w + i * _GBLK
        pltpu.sync_copy(idx_hbm.at[pl.ds(base, _GBLK)], idx_v)
        pltpu.async_copy(table_hbm.at[idx_v], rows_v, sem).wait()
        pltpu.sync_copy(rows_v, out_hbm.at[pl.ds(base, _GBLK)])
        return carry
    lax.fori_loop(0, per_w // _GBLK, body, 0)


def _gather_rows(table, idx):
    d = table.shape[1]
    mesh = plsc.VectorSubcoreMesh(core_axis_name="c", subcore_axis_name="s")
    kern = functools.partial(
        pl.kernel, mesh=mesh,
        out_type=jax.ShapeDtypeStruct((N * K, d), jnp.float32),
        scratch_types=[
            pltpu.VMEM((_GBLK,), jnp.int32),
            pltpu.VMEM((_GBLK, d), jnp.float32),
            pltpu.SemaphoreType.DMA,
        ],
    )(_gather_body)
    return kern(table, idx)


_GR = 80


def _graph_kernel(pos_ref, bat_ref, post_ref, batf_ref, nb_ref, val_ref):
    p = pos_ref[...]
    pt = post_ref[...]
    sq_r = jnp.sum(p * p, axis=1)
    sq_c = jnp.sum(pt * pt, axis=0)
    cross = jax.lax.dot_general(p, pt, (((1,), (0,)), ((), ())),
                                preferred_element_type=jnp.float32)
    d2 = sq_r[:, None] + sq_c[None, :] - 2.0 * cross
    same = bat_ref[...] == batf_ref[...]
    score = jnp.where(same & (d2 <= 1.0), d2, jnp.inf)
    cols = lax.broadcasted_iota(jnp.int32, (_GR, N), 1)
    big = jnp.int32(2 ** 30)

    def sel(k, carry):
        sc, nb, vl = carry
        m = jnp.min(sc, axis=1)
        hit = sc == m[:, None]
        idx = jnp.min(jnp.where(hit, cols, big), axis=1)
        kcol = lax.broadcasted_iota(jnp.int32, (_GR, K), 1) == k
        ok = jnp.isfinite(m)
        idx_safe = jnp.where(ok, idx, 0)
        nb = jnp.where(kcol, idx_safe[:, None], nb)
        vl = jnp.where(kcol, ok[:, None].astype(jnp.float32), vl)
        sc = jnp.where(cols == idx[:, None], jnp.inf, sc)
        return sc, nb, vl

    nb0 = jnp.zeros((_GR, K), jnp.int32)
    vl0 = jnp.zeros((_GR, K), jnp.float32)
    _, nb, vl = lax.fori_loop(0, K, sel, (score, nb0, vl0))
    nb_ref[...] = nb
    val_ref[...] = vl


def _graph(pos, batch):
    grid = (N // _GR,)
    return pl.pallas_call(
        _graph_kernel,
        grid=grid,
        in_specs=[
            pl.BlockSpec((_GR, 2), lambda i: (i, 0)),
            pl.BlockSpec((_GR, 1), lambda i: (i, 0)),
            pl.BlockSpec((2, N), lambda i: (0, 0)),
            pl.BlockSpec((1, N), lambda i: (0, 0)),
        ],
        out_specs=[
            pl.BlockSpec((_GR, K), lambda i: (i, 0)),
            pl.BlockSpec((_GR, K), lambda i: (i, 0)),
        ],
        out_shape=[
            jax.ShapeDtypeStruct((N, K), jnp.int32),
            jax.ShapeDtypeStruct((N, K), jnp.float32),
        ],
    )(pos, batch.reshape(N, 1), pos.T, batch.reshape(1, N))


_TN = 200


def _pre_kernel(x_ref, pos_ref, part_ref, wx_ref, wp_ref, b_ref,
                u_ref, q_ref, c_ref):
    part = part_ref[...]
    den = jnp.sum(part[:, 0:16] + part[:, 128:144], axis=1) + 1e-30
    nx = jnp.sum(part[:, 16:32] + part[:, 144:160], axis=1)
    ny = jnp.sum(part[:, 32:48] + part[:, 160:176], axis=1)
    c_ref[...] = jnp.stack([nx / den, ny / den], axis=1)
    pos = pos_ref[...]
    q = jax.lax.dot_general(pos, wp_ref[...], (((1,), (0,)), ((), ())),
                            preferred_element_type=jnp.float32)
    u = jax.lax.dot_general(x_ref[...], wx_ref[...], (((1,), (0,)), ((), ())),
                            preferred_element_type=jnp.float32)
    q_ref[...] = q
    u_ref[...] = u + q + b_ref[...]


def _pre(x, pos, partials, wx, wp, b):
    fh = wx.shape[1]
    grid = (N // _TN,)
    return pl.pallas_call(
        _pre_kernel,
        grid=grid,
        in_specs=[
            pl.BlockSpec((_TN, 256), lambda i: (i, 0)),
            pl.BlockSpec((_TN, 2), lambda i: (i, 0)),
            pl.BlockSpec((_TN, 2 * 128), lambda i: (i, 0)),
            pl.BlockSpec((256, fh), lambda i: (0, 0)),
            pl.BlockSpec((2, fh), lambda i: (0, 0)),
            pl.BlockSpec((1, fh), lambda i: (0, 0)),
        ],
        out_specs=[
            pl.BlockSpec((_TN, fh), lambda i: (i, 0)),
            pl.BlockSpec((_TN, fh), lambda i: (i, 0)),
            pl.BlockSpec((_TN, 2), lambda i: (i, 0)),
        ],
        out_shape=[
            jax.ShapeDtypeStruct((N, fh), jnp.float32),
            jax.ShapeDtypeStruct((N, fh), jnp.float32),
            jax.ShapeDtypeStruct((N, 2), jnp.float32),
        ],
    )(x, pos, partials, wx, wp, b)


def _dot(a, b):
    return jax.lax.dot_general(a, b, (((1,), (0,)), ((), ())),
                               preferred_element_type=jnp.float32)


def _edge_and_node(us_ref, q_ref, x_ref, val_ref,
                   wf2_ref, bf2_ref, wg1_ref, bg1_ref, wg2_ref, bg2_ref):
    fh = q_ref.shape[1]
    fo = wf2_ref.shape[1]
    us = us_ref[...]
    q = q_ref[...]
    h = jax.nn.relu(us - q[:, None, :]).reshape(_TN * K, fh)
    h2 = jax.nn.relu(_dot(h, wf2_ref[...]) + bf2_ref[...])
    h3 = h2.reshape(_TN, K, fo) * val_ref[...][:, :, None]
    agg = jnp.sum(h3, axis=1)
    x = x_ref[...]
    xd = x.shape[1]
    g1 = jax.nn.relu(_dot(x, wg1_ref[:xd, :]) + _dot(agg, wg1_ref[xd:, :])
                     + bg1_ref[...])
    return _dot(g1, wg2_ref[...]) + bg2_ref[...]


def _mid_layer_kernel(us_ref, q_ref, x_ref, pos_ref, val_ref,
                      wf2_ref, bf2_ref, wg1_ref, bg1_ref, wg2_ref, bg2_ref,
                      wxn_ref, wpn_ref, bn_ref,
                      xo_ref, un_ref, qn_ref):
    xo = _edge_and_node(us_ref, q_ref, x_ref, val_ref,
                        wf2_ref, bf2_ref, wg1_ref, bg1_ref, wg2_ref, bg2_ref)
    xo_ref[...] = xo
    qn = _dot(pos_ref[...], wpn_ref[...])
    qn_ref[...] = qn
    un_ref[...] = _dot(xo, wxn_ref[...]) + qn + bn_ref[...]


def _last_layer_kernel(us_ref, q_ref, x_ref, val_ref, luv_ref, tinv_ref,
                       wf2_ref, bf2_ref, wg1_ref, bg1_ref, wg2_ref, bg2_ref,
                       lzp_ref, logit_ref):
    xo = _edge_and_node(us_ref, q_ref, x_ref, val_ref,
                        wf2_ref, bf2_ref, wg1_ref, bg1_ref, wg2_ref, bg2_ref)
    logit = 8.8 * jnp.tanh(xo)
    logit_ref[...] = logit
    z = (logit + luv_ref[...]) * tinv_ref[...]
    lzp_ref[...] = jax.nn.log_sigmoid(z)


def _full(r, c):
    return pl.BlockSpec((r, c), lambda i: (0, 0))


def _rows(c):
    return pl.BlockSpec((_TN, c), lambda i: (i, 0))


def _mid_layer(us, q, x, pos, valid, wf2, bf2, wg1, bg1, wg2, bg2,
               wxn, wpn, bn):
    fh = q.shape[1]
    fo = wf2.shape[1]
    gh = wg1.shape[1]
    go = wg2.shape[1]
    fhn = wxn.shape[1]
    xd = x.shape[1]
    grid = (N // _TN,)
    return pl.pallas_call(
        _mid_layer_kernel,
        grid=grid,
        in_specs=[
            pl.BlockSpec((_TN, K, fh), lambda i: (i, 0, 0)),
            _rows(fh), _rows(xd), _rows(2), _rows(K),
            _full(fh, fo), _full(1, fo),
            _full(xd + fo, gh), _full(1, gh),
            _full(gh, go), _full(1, go),
            _full(go, fhn), _full(2, fhn), _full(1, fhn),
        ],
        out_specs=[_rows(go), _rows(fhn), _rows(fhn)],
        out_shape=[
            jax.ShapeDtypeStruct((N, go), jnp.float32),
            jax.ShapeDtypeStruct((N, fhn), jnp.float32),
            jax.ShapeDtypeStruct((N, fhn), jnp.float32),
        ],
    )(us, q, x, pos, valid, wf2, bf2, wg1, bg1, wg2, bg2, wxn, wpn, bn)


def _last_layer(us, q, x, valid, luv, tinv, wf2, bf2, wg1, bg1, wg2, bg2):
    fh = q.shape[1]
    fo = wf2.shape[1]
    gh = wg1.shape[1]
    xd = x.shape[1]
    grid = (N // _TN,)
    return pl.pallas_call(
        _last_layer_kernel,
        grid=grid,
        in_specs=[
            pl.BlockSpec((_TN, K, fh), lambda i: (i, 0, 0)),
            _rows(fh), _rows(xd), _rows(K), _rows(1), _full(1, 1),
            _full(fh, fo), _full(1, fo),
            _full(xd + fo, gh), _full(1, gh),
            _full(gh, 1), _full(1, 1),
        ],
        out_specs=[_rows(1), _rows(1)],
        out_shape=[
            jax.ShapeDtypeStruct((N, 1), jnp.float32),
            jax.ShapeDtypeStruct((N, 1), jnp.float32),
        ],
    )(us, q, x, valid, luv, tinv, wf2, bf2, wg1, bg1, wg2, bg2)


def kernel(glimpse__feature, glimpse__center, glimpse__batch,
           glimpse_member__local_pos, glimpse_member__log_mask,
           glimpse_member__glimpse_index, temperature,
           w1f1, b1f1, w1f2, b1f2, w1g1, b1g1, w1g2, b1g2,
           w2f1, b2f1, w2f2, b2f2, w2g1, b2g1, w2g2, b2g2,
           w3f1, b3f1, w3f2, b3f2, w3g1, b3g1, w3g2, b3g2,
           zp_w, zp_b):
    x = glimpse__feature
    pos = glimpse__center / RADIUS_MAX
    batch = glimpse__batch.astype(jnp.int32)

    gidx = glimpse_member__glimpse_index.astype(jnp.int32)
    lm = glimpse_member__log_mask.reshape(M)
    px = glimpse_member__local_pos[:, 0]
    py = glimpse_member__local_pos[:, 1]
    partials = _member_segsum(gidx, lm, px, py)
    partials = partials.reshape(NC, _NPAD, 128)[:, :N, :]
    partials = partials.transpose(1, 0, 2).reshape(N, 256)

    nb, valid = _graph(pos, batch)
    nbflat = nb.reshape(N * K)

    u1, q1, center = _pre(x, pos, partials,
                          w1f1[:256, :], w1f1[256:, :], b1f1.reshape(1, -1))

    us1 = _gather_rows(u1, nbflat).reshape(N, K, -1)
    x1, u2, q2 = _mid_layer(us1, q1, x, pos, valid,
                            w1f2, b1f2.reshape(1, -1),
                            w1g1, b1g1.reshape(1, -1),
                            w1g2, b1g2.reshape(1, -1),
                            w2f1[:64, :], w2f1[64:, :], b2f1.reshape(1, -1))

    u2p = jnp.pad(u2, ((0, 0), (0, 128 - u2.shape[1])))
    us2 = _gather_rows(u2p, nbflat)[:, :u2.shape[1]].reshape(N, K, -1)
    x2, u3, q3 = _mid_layer(us2, q2, x1, pos, valid,
                            w2f2, b2f2.reshape(1, -1),
                            w2g1, b2g1.reshape(1, -1),
                            w2g2, b2g2.reshape(1, -1),
                            w3f1[:32, :], w3f1[32:, :], b3f1.reshape(1, -1))

    u3p = jnp.pad(u3, ((0, 0), (0, 128 - u3.shape[1])))
    us3 = _gather_rows(u3p, nbflat)[:, :u3.shape[1]].reshape(N, K, -1)
    wg2z = w3g2 @ zp_w
    bg2z = (b3g2 @ zp_w + zp_b).reshape(1, 1)
    u = jax.random.uniform(jax.random.key(123), (N,), jnp.float32,
                           1e-6, 1.0 - 1e-6)
    luv = (jnp.log(u) - jnp.log1p(-u)).reshape(N, 1)
    tinv = (1.0 / temperature).reshape(1, 1).astype(jnp.float32)
    lzp, logit = _last_layer(us3, q3, x2, valid, luv, tinv,
                             w3f2, b3f2.reshape(1, -1),
                             w3g1, b3g1.reshape(1, -1),
                             wg2z, bg2z)
    return (lzp.reshape(N), logit.reshape(N), center)


# speedup vs baseline: 6.5026x; 2.0031x over previous
import functools
import jax
import jax.numpy as jnp
from jax import lax
from jax.experimental import pallas as pl
from jax.experimental.pallas import tpu as pltpu
from jax.experimental.pallas import tpu_sc as plsc

N = 10000
M = 160000
K = 32
RADIUS_MAX = 4.0

NC = 2
NS = 16
NW = NC * NS


_MBLK = 640
_NBLK = M // _MBLK
_CHW = 16


_NPAD = 10240
_NSTRIPE = _NPAD // NS


def _member_body(gidx_hbm, lm_hbm, px_hbm, py_hbm, zeros_hbm, out_hbm,
                 gv, lv, xv, yv, ix, vals, shared):
    cid = lax.axis_index("c")
    sid = lax.axis_index("s")
    wid = sid * NC + cid

    pltpu.sync_copy(zeros_hbm.at[pl.ds(sid * _NSTRIPE, _NSTRIPE)],
                    shared.at[pl.ds(sid * _NSTRIPE, _NSTRIPE)])
    plsc.subcore_barrier()

    pltpu.sync_copy(zeros_hbm.at[pl.ds(0, 16)], vals)

    lanes = lax.iota(jnp.int32, 16)
    zf = jnp.zeros((16,), jnp.float32)

    def blk_body(bi, carry):
        b = wid + bi * NW

        @pl.when(b < _NBLK)
        def _():
            base = b * _MBLK
            pltpu.sync_copy(gidx_hbm.at[pl.ds(base, _MBLK)], gv)
            pltpu.sync_copy(lm_hbm.at[pl.ds(base, _MBLK)], lv)
            pltpu.sync_copy(px_hbm.at[pl.ds(base, _MBLK)], xv)
            pltpu.sync_copy(py_hbm.at[pl.ds(base, _MBLK)], yv)

            def chunk_body(j, c2):
                sl = pl.ds(j * 16, 16)
                g = gv[sl]
                w = jnp.exp(lv[sl])
                wx = w * xv[sl]
                wy = w * yv[sl]
                ix[...] = g
                for r in range(16):
                    hit = lanes == r
                    vals[r, pl.ds(0, 16)] = jnp.where(hit, w, zf)
                    vals[r, pl.ds(16, 16)] = jnp.where(hit, wx, zf)
                    vals[r, pl.ds(32, 16)] = jnp.where(hit, wy, zf)
                pltpu.sync_copy(vals, shared.at[ix], add=True)
                return c2
            lax.fori_loop(0, _MBLK // 16, chunk_body, 0)
        return carry

    lax.fori_loop(0, (_NBLK + NW - 1) // NW, blk_body, 0)
    plsc.subcore_barrier()
    pltpu.sync_copy(shared.at[pl.ds(sid * _NSTRIPE, _NSTRIPE)],
                    out_hbm.at[pl.ds(cid * _NPAD + sid * _NSTRIPE, _NSTRIPE)])


def _member_segsum(gidx, lm, px, py):
    mesh = plsc.VectorSubcoreMesh(core_axis_name="c", subcore_axis_name="s")
    zeros = jnp.zeros((_NPAD, 128), jnp.float32)
    kern = functools.partial(
        pl.kernel, mesh=mesh,
        out_type=jax.ShapeDtypeStruct((NC * _NPAD, 128), jnp.float32),
        scratch_types=[
            pltpu.VMEM((_MBLK,), jnp.int32),
            pltpu.VMEM((_MBLK,), jnp.float32),
            pltpu.VMEM((_MBLK,), jnp.float32),
            pltpu.VMEM((_MBLK,), jnp.float32),
            pltpu.VMEM((16,), jnp.int32),
            pltpu.VMEM((16, 128), jnp.float32),
            pltpu.VMEM_SHARED((_NPAD, 128), jnp.float32),
        ],
    )(_member_body)
    return kern(gidx, lm, px, py, zeros)


_GBLK = 80


def _gather_body(table_hbm, idx_hbm, out_hbm, idx_v, rows_v, sem):
    wid = lax.axis_index("s") * NC + lax.axis_index("c")
    per_w = (N * K) // NW

    def body(i, carry):
        base = wid * per_w + i * _GBLK
        pltpu.sync_copy(idx_hbm.at[pl.ds(base, _GBLK)], idx_v)
        pltpu.async_copy(table_hbm.at[idx_v], rows_v, sem).wait()
        pltpu.sync_copy(rows_v, out_hbm.at[pl.ds(base, _GBLK)])
        return carry
    lax.fori_loop(0, per_w // _GBLK, body, 0)


def _gather_rows(table, idx):
    d = table.shape[1]
    mesh = plsc.VectorSubcoreMesh(core_axis_name="c", subcore_axis_name="s")
    kern = functools.partial(
        pl.kernel, mesh=mesh,
        out_type=jax.ShapeDtypeStruct((N * K, d), jnp.float32),
        scratch_types=[
            pltpu.VMEM((_GBLK,), jnp.int32),
            pltpu.VMEM((_GBLK, d), jnp.float32),
            pltpu.SemaphoreType.DMA,
        ],
    )(_gather_body)
    return kern(table, idx)


_GR = 80


_WB = 1024
_W = 4096
_WPAD = 13312


def _graph_kernel(s_ref, pos_ref, bat_ref, pt0, pt1, pt2, pt3,
                  bf0, bf1, bf2, bf3, nb_ref, val_ref):
    p = pos_ref[...]
    pt = jnp.concatenate([pt0[...], pt1[...], pt2[...], pt3[...]], axis=1)
    bf = jnp.concatenate([bf0[...], bf1[...], bf2[...], bf3[...]], axis=1)
    sq_r = jnp.sum(p * p, axis=1)
    sq_c = jnp.sum(pt * pt, axis=0)
    cross = jax.lax.dot_general(p, pt, (((1,), (0,)), ((), ())),
                                preferred_element_type=jnp.float32)
    d2 = sq_r[:, None] + sq_c[None, :] - 2.0 * cross
    same = bat_ref[...] == bf
    score = jnp.where(same & (d2 <= 1.0), d2, jnp.inf)
    cols = lax.broadcasted_iota(jnp.int32, (_GR, _W), 1)
    base = s_ref[pl.program_id(0)] * _WB
    big = jnp.int32(2 ** 30)

    def sel(k, carry):
        sc, nb, vl = carry
        m = jnp.min(sc, axis=1)
        hit = sc == m[:, None]
        idx = jnp.min(jnp.where(hit, cols, big), axis=1)
        kcol = lax.broadcasted_iota(jnp.int32, (_GR, K), 1) == k
        ok = jnp.isfinite(m)
        idx_safe = jnp.where(ok, idx + base, 0)
        nb = jnp.where(kcol, idx_safe[:, None], nb)
        vl = jnp.where(kcol, ok[:, None].astype(jnp.float32), vl)
        sc = jnp.where(cols == idx[:, None], jnp.inf, sc)
        return sc, nb, vl

    nb0 = jnp.zeros((_GR, K), jnp.int32)
    vl0 = jnp.zeros((_GR, K), jnp.float32)
    _, nb, vl = lax.fori_loop(0, K, sel, (score, nb0, vl0))
    nb_ref[...] = nb
    val_ref[...] = vl


def _graph(pos, batch):
    grid = (N // _GR,)
    b2 = batch.reshape(N // _GR, _GR)
    first = jnp.searchsorted(batch, b2[:, 0], side='left').astype(jnp.int32)
    s = jnp.clip(first // _WB, 0, (_WPAD - _W) // _WB)
    post = jnp.concatenate(
        [pos.T, jnp.full((2, _WPAD - N), 1e6, jnp.float32)], axis=1)
    batf = jnp.concatenate(
        [batch.reshape(1, N), jnp.full((1, _WPAD - N), -1, jnp.int32)], axis=1)
    pspec = [pl.BlockSpec((2, _WB), (lambda o: (lambda i, s: (0, s[i] + o)))(o))
             for o in range(4)]
    bspec = [pl.BlockSpec((1, _WB), (lambda o: (lambda i, s: (0, s[i] + o)))(o))
             for o in range(4)]
    return pl.pallas_call(
        _graph_kernel,
        grid_spec=pltpu.PrefetchScalarGridSpec(
            num_scalar_prefetch=1,
            grid=grid,
            in_specs=[
                pl.BlockSpec((_GR, 2), lambda i, s: (i, 0)),
                pl.BlockSpec((_GR, 1), lambda i, s: (i, 0)),
            ] + pspec + bspec,
            out_specs=[
                pl.BlockSpec((_GR, K), lambda i, s: (i, 0)),
                pl.BlockSpec((_GR, K), lambda i, s: (i, 0)),
            ],
        ),
        out_shape=[
            jax.ShapeDtypeStruct((N, K), jnp.int32),
            jax.ShapeDtypeStruct((N, K), jnp.float32),
        ],
    )(s, pos, batch.reshape(N, 1), post, post, post, post,
      batf, batf, batf, batf)


_TN = 200


def _pre_kernel(x_ref, pos_ref, part_ref, wx_ref, wp_ref, b_ref,
                u_ref, q_ref, c_ref):
    part = part_ref[...]
    den = jnp.sum(part[:, 0:16] + part[:, 128:144], axis=1) + 1e-30
    nx = jnp.sum(part[:, 16:32] + part[:, 144:160], axis=1)
    ny = jnp.sum(part[:, 32:48] + part[:, 160:176], axis=1)
    c_ref[...] = jnp.stack([nx / den, ny / den], axis=1)
    pos = pos_ref[...]
    q = jax.lax.dot_general(pos, wp_ref[...], (((1,), (0,)), ((), ())),
                            preferred_element_type=jnp.float32)
    u = jax.lax.dot_general(x_ref[...], wx_ref[...], (((1,), (0,)), ((), ())),
                            preferred_element_type=jnp.float32)
    q_ref[...] = q
    u_ref[...] = u + q + b_ref[...]


def _pre(x, pos, partials, wx, wp, b):
    fh = wx.shape[1]
    grid = (N // _TN,)
    return pl.pallas_call(
        _pre_kernel,
        grid=grid,
        in_specs=[
            pl.BlockSpec((_TN, 256), lambda i: (i, 0)),
            pl.BlockSpec((_TN, 2), lambda i: (i, 0)),
            pl.BlockSpec((_TN, 2 * 128), lambda i: (i, 0)),
            pl.BlockSpec((256, fh), lambda i: (0, 0)),
            pl.BlockSpec((2, fh), lambda i: (0, 0)),
            pl.BlockSpec((1, fh), lambda i: (0, 0)),
        ],
        out_specs=[
            pl.BlockSpec((_TN, fh), lambda i: (i, 0)),
            pl.BlockSpec((_TN, fh), lambda i: (i, 0)),
            pl.BlockSpec((_TN, 2), lambda i: (i, 0)),
        ],
        out_shape=[
            jax.ShapeDtypeStruct((N, fh), jnp.float32),
            jax.ShapeDtypeStruct((N, fh), jnp.float32),
            jax.ShapeDtypeStruct((N, 2), jnp.float32),
        ],
    )(x, pos, partials, wx, wp, b)


def _dot(a, b):
    return jax.lax.dot_general(a, b, (((1,), (0,)), ((), ())),
                               preferred_element_type=jnp.float32)


def _edge_and_node(us_ref, q_ref, x_ref, val_ref,
                   wf2_ref, bf2_ref, wg1_ref, bg1_ref, wg2_ref, bg2_ref):
    fh = q_ref.shape[1]
    fo = wf2_ref.shape[1]
    us = us_ref[...]
    q = q_ref[...]
    h = jax.nn.relu(us - q[:, None, :]).reshape(_TN * K, fh)
    h2 = jax.nn.relu(_dot(h, wf2_ref[...]) + bf2_ref[...])
    h3 = h2.reshape(_TN, K, fo) * val_ref[...][:, :, None]
    agg = jnp.sum(h3, axis=1)
    x = x_ref[...]
    xd = x.shape[1]
    g1 = jax.nn.relu(_dot(x, wg1_ref[:xd, :]) + _dot(agg, wg1_ref[xd:, :])
                     + bg1_ref[...])
    return _dot(g1, wg2_ref[...]) + bg2_ref[...]


def _mid_layer_kernel(us_ref, q_ref, x_ref, pos_ref, val_ref,
                      wf2_ref, bf2_ref, wg1_ref, bg1_ref, wg2_ref, bg2_ref,
                      wxn_ref, wpn_ref, bn_ref,
                      xo_ref, un_ref, qn_ref):
    xo = _edge_and_node(us_ref, q_ref, x_ref, val_ref,
                        wf2_ref, bf2_ref, wg1_ref, bg1_ref, wg2_ref, bg2_ref)
    xo_ref[...] = xo
    qn = _dot(pos_ref[...], wpn_ref[...])
    qn_ref[...] = qn
    un_ref[...] = _dot(xo, wxn_ref[...]) + qn + bn_ref[...]


def _last_layer_kernel(us_ref, q_ref, x_ref, val_ref, luv_ref, tinv_ref,
                       wf2_ref, bf2_ref, wg1_ref, bg1_ref, wg2_ref, bg2_ref,
                       lzp_ref, logit_ref):
    xo = _edge_and_node(us_ref, q_ref, x_ref, val_ref,
                        wf2_ref, bf2_ref, wg1_ref, bg1_ref, wg2_ref, bg2_ref)
    logit = 8.8 * jnp.tanh(xo)
    logit_ref[...] = logit
    z = (logit + luv_ref[...]) * tinv_ref[...]
    lzp_ref[...] = jax.nn.log_sigmoid(z)


def _full(r, c):
    return pl.BlockSpec((r, c), lambda i: (0, 0))


def _rows(c):
    return pl.BlockSpec((_TN, c), lambda i: (i, 0))


def _mid_layer(us, q, x, pos, valid, wf2, bf2, wg1, bg1, wg2, bg2,
               wxn, wpn, bn):
    fh = q.shape[1]
    fo = wf2.shape[1]
    gh = wg1.shape[1]
    go = wg2.shape[1]
    fhn = wxn.shape[1]
    xd = x.shape[1]
    grid = (N // _TN,)
    return pl.pallas_call(
        _mid_layer_kernel,
        grid=grid,
        in_specs=[
            pl.BlockSpec((_TN, K, fh), lambda i: (i, 0, 0)),
            _rows(fh), _rows(xd), _rows(2), _rows(K),
            _full(fh, fo), _full(1, fo),
            _full(xd + fo, gh), _full(1, gh),
            _full(gh, go), _full(1, go),
            _full(go, fhn), _full(2, fhn), _full(1, fhn),
        ],
        out_specs=[_rows(go), _rows(fhn), _rows(fhn)],
        out_shape=[
            jax.ShapeDtypeStruct((N, go), jnp.float32),
            jax.ShapeDtypeStruct((N, fhn), jnp.float32),
            jax.ShapeDtypeStruct((N, fhn), jnp.float32),
        ],
    )(us, q, x, pos, valid, wf2, bf2, wg1, bg1, wg2, bg2, wxn, wpn, bn)


def _last_layer(us, q, x, valid, luv, tinv, wf2, bf2, wg1, bg1, wg2, bg2):
    fh = q.shape[1]
    fo = wf2.shape[1]
    gh = wg1.shape[1]
    xd = x.shape[1]
    grid = (N // _TN,)
    return pl.pallas_call(
        _last_layer_kernel,
        grid=grid,
        in_specs=[
            pl.BlockSpec((_TN, K, fh), lambda i: (i, 0, 0)),
            _rows(fh), _rows(xd), _rows(K), _rows(1), _full(1, 1),
            _full(fh, fo), _full(1, fo),
            _full(xd + fo, gh), _full(1, gh),
            _full(gh, 1), _full(1, 1),
        ],
        out_specs=[_rows(1), _rows(1)],
        out_shape=[
            jax.ShapeDtypeStruct((N, 1), jnp.float32),
            jax.ShapeDtypeStruct((N, 1), jnp.float32),
        ],
    )(us, q, x, valid, luv, tinv, wf2, bf2, wg1, bg1, wg2, bg2)


def kernel(glimpse__feature, glimpse__center, glimpse__batch,
           glimpse_member__local_pos, glimpse_member__log_mask,
           glimpse_member__glimpse_index, temperature,
           w1f1, b1f1, w1f2, b1f2, w1g1, b1g1, w1g2, b1g2,
           w2f1, b2f1, w2f2, b2f2, w2g1, b2g1, w2g2, b2g2,
           w3f1, b3f1, w3f2, b3f2, w3g1, b3g1, w3g2, b3g2,
           zp_w, zp_b):
    x = glimpse__feature
    pos = glimpse__center / RADIUS_MAX
    batch = glimpse__batch.astype(jnp.int32)

    gidx = glimpse_member__glimpse_index.astype(jnp.int32)
    lm = glimpse_member__log_mask.reshape(M)
    px = glimpse_member__local_pos[:, 0]
    py = glimpse_member__local_pos[:, 1]
    partials = _member_segsum(gidx, lm, px, py)
    partials = partials.reshape(NC, _NPAD, 128)[:, :N, :]
    partials = partials.transpose(1, 0, 2).reshape(N, 256)

    nb, valid = _graph(pos, batch)
    nbflat = nb.reshape(N * K)

    u1, q1, center = _pre(x, pos, partials,
                          w1f1[:256, :], w1f1[256:, :], b1f1.reshape(1, -1))

    us1 = _gather_rows(u1, nbflat).reshape(N, K, -1)
    x1, u2, q2 = _mid_layer(us1, q1, x, pos, valid,
                            w1f2, b1f2.reshape(1, -1),
                            w1g1, b1g1.reshape(1, -1),
                            w1g2, b1g2.reshape(1, -1),
                            w2f1[:64, :], w2f1[64:, :], b2f1.reshape(1, -1))

    u2p = jnp.pad(u2, ((0, 0), (0, 128 - u2.shape[1])))
    us2 = _gather_rows(u2p, nbflat)[:, :u2.shape[1]].reshape(N, K, -1)
    x2, u3, q3 = _mid_layer(us2, q2, x1, pos, valid,
                            w2f2, b2f2.reshape(1, -1),
                            w2g1, b2g1.reshape(1, -1),
                            w2g2, b2g2.reshape(1, -1),
                            w3f1[:32, :], w3f1[32:, :], b3f1.reshape(1, -1))

    u3p = jnp.pad(u3, ((0, 0), (0, 128 - u3.shape[1])))
    us3 = _gather_rows(u3p, nbflat)[:, :u3.shape[1]].reshape(N, K, -1)
    wg2z = w3g2 @ zp_w
    bg2z = (b3g2 @ zp_w + zp_b).reshape(1, 1)
    u = jax.random.uniform(jax.random.key(123), (N,), jnp.float32,
                           1e-6, 1.0 - 1e-6)
    luv = (jnp.log(u) - jnp.log1p(-u)).reshape(N, 1)
    tinv = (1.0 / temperature).reshape(1, 1).astype(jnp.float32)
    lzp, logit = _last_layer(us3, q3, x2, valid, luv, tinv,
                             w3f2, b3f2.reshape(1, -1),
                             w3g1, b3g1.reshape(1, -1),
                             wg2z, bg2z)
    return (lzp.reshape(N), logit.reshape(N), center)
